# baseline (speedup 1.0000x reference)
import jax
import jax.numpy as jnp
from jax.experimental import pallas as pl


def _cmp_body(lab_ref, out_ref, *, blk):
    i = pl.program_id(0)
    cls = jax.lax.broadcasted_iota(jnp.int32, (blk, 1), 0) + i * blk
    out_ref[...] = (lab_ref[...] == cls).astype(jnp.float32)


def kernel(label, N):
    n_cls = 1000
    b, l = label.shape
    flat = label.reshape(1, b * l)
    blk = 8
    import functools
    out = pl.pallas_call(
        functools.partial(_cmp_body, blk=blk),
        grid=(n_cls // blk,),
        in_specs=[pl.BlockSpec((1, b * l), lambda i: (0, 0))],
        out_specs=pl.BlockSpec((blk, b * l), lambda i: (i, 0)),
        out_shape=jax.ShapeDtypeStruct((n_cls, b * l), jnp.float32),
    )(flat)
    return out.reshape(n_cls, b, l)


# speedup vs baseline: 1.0060x; 1.0060x over previous
import functools

import jax
import jax.numpy as jnp
from jax.experimental import pallas as pl
from jax.experimental.pallas import tpu as pltpu

_BLK = 8
_NBUF = 8


def _body(lab_ref, out_ref, scratch, sems, *, n_cls, width):
    i = pl.program_id(0)
    ngrid = n_cls // _BLK
    slot = jax.lax.rem(i, _NBUF)

    @pl.when(i >= _NBUF)
    def _wait_prev():
        j = i - _NBUF
        pltpu.make_async_copy(
            scratch.at[pl.ds(slot * _BLK, _BLK)],
            out_ref.at[pl.ds(j * _BLK, _BLK)],
            sems.at[slot],
        ).wait()

    cls = jax.lax.broadcasted_iota(jnp.int32, (_BLK, 1), 0) + i * _BLK
    scratch[pl.ds(slot * _BLK, _BLK), :] = (lab_ref[...] == cls).astype(
        jnp.float32
    )
    pltpu.make_async_copy(
        scratch.at[pl.ds(slot * _BLK, _BLK)],
        out_ref.at[pl.ds(i * _BLK, _BLK)],
        sems.at[slot],
    ).start()

    @pl.when(i == ngrid - 1)
    def _drain():
        for k in range(_NBUF):
            j = ngrid - _NBUF + k
            s = j % _NBUF
            pltpu.make_async_copy(
                scratch.at[pl.ds(s * _BLK, _BLK)],
                out_ref.at[pl.ds(j * _BLK, _BLK)],
                sems.at[s],
            ).wait()


def kernel(label, N):
    n_cls = 1000
    b, l = label.shape
    width = b * l
    flat = label.reshape(1, width)
    out = pl.pallas_call(
        functools.partial(_body, n_cls=n_cls, width=width),
        grid=(n_cls // _BLK,),
        in_specs=[pl.BlockSpec((1, width), lambda i: (0, 0))],
        out_specs=pl.BlockSpec(memory_space=pltpu.MemorySpace.HBM),
        out_shape=jax.ShapeDtypeStruct((n_cls, width), jnp.float32),
        scratch_shapes=[
            pltpu.VMEM((_NBUF * _BLK, width), jnp.float32),
            pltpu.SemaphoreType.DMA((_NBUF,)),
        ],
    )(flat)
    return out.reshape(n_cls, b, l)


# speedup vs baseline: 9.6067x; 9.5492x over previous
import functools

import jax
import jax.numpy as jnp
from jax.experimental import pallas as pl

_BLKC = 200


def _body(lab_ref, out_ref):
    cb = pl.program_id(1)
    cls = jax.lax.broadcasted_iota(jnp.int32, (_BLKC, 1), 0) + cb * _BLKC
    out_ref[0] = (lab_ref[0] == cls).astype(jnp.float32)


def kernel(label, N):
    n_cls = 1000
    b, l = label.shape
    lab_t = label.T.reshape(l, 1, b)
    out = pl.pallas_call(
        _body,
        grid=(l, n_cls // _BLKC),
        in_specs=[pl.BlockSpec((1, 1, b), lambda j, cb: (j, 0, 0))],
        out_specs=pl.BlockSpec((1, _BLKC, b), lambda j, cb: (j, cb, 0)),
        out_shape=jax.ShapeDtypeStruct((l, n_cls, b), jnp.float32),
    )(lab_t)
    return out.transpose(1, 2, 0)
